# baseline (speedup 1.0000x reference)
import functools

import jax
import jax.numpy as jnp
from jax import lax
from jax.experimental import pallas as pl
from jax.experimental.pallas import tpu as pltpu
from jax.experimental.pallas import tpu_sc as plsc

B, F, V, D = 16384, 26, 8, 128
NC, NS = 2, 16
NW = NC * NS
ROWS = B * F
RPW = ROWS // NW
CH = 128
NCH = RPW // CH
NB = 4
NBANDS = B // 8


@functools.partial(
    pl.kernel,
    out_type=jax.ShapeDtypeStruct((NBANDS, F, 8, D), jnp.float32),
    mesh=plsc.VectorSubcoreMesh(core_axis_name="c", subcore_axis_name="s"),
    scratch_types=(
        [pltpu.VMEM((NCH, CH), jnp.int32)]
        + [pltpu.VMEM((CH, D), jnp.float32) for _ in range(NB)]
        + [pltpu.VMEM_SHARED((4 * F * V, D), jnp.float32)]
        + [pltpu.SemaphoreType.DMA for _ in range(2 * NB)]
    ),
)
def _sc_lookup(tbl_hbm, idx_hbm, out_4d, idx_v, *rest):
    out_hbm = out_4d.reshape(ROWS, D)
    bufs = rest[:NB]
    tbl_sh = rest[NB]
    gsem = rest[NB + 1:2 * NB + 1]
    ssem = rest[2 * NB + 1:]

    wid = lax.axis_index("s") * NC + lax.axis_index("c")

    @pl.when(lax.axis_index("s") == 0)
    def _stage_table():
        pltpu.sync_copy(tbl_hbm, tbl_sh)

    pltpu.sync_copy(idx_hbm.at[wid], idx_v)
    plsc.subcore_barrier()

    base = wid * RPW

    def start_gather(g, p):
        pltpu.async_copy(tbl_sh.at[idx_v.at[g]], bufs[p], gsem[p])

    def wait_gather(g, p):
        pltpu.make_async_copy(tbl_sh.at[idx_v.at[g]], bufs[p], gsem[p]).wait()

    def start_scatter(g, p):
        pltpu.async_copy(bufs[p], out_hbm.at[pl.ds(base + g * CH, CH)], ssem[p])

    def wait_scatter(g, p):
        pltpu.make_async_copy(bufs[p], out_hbm.at[pl.ds(base + g * CH, CH)],
                              ssem[p]).wait()

    for p in range(NB):
        start_gather(p, p)

    def body(k, carry):
        g = NB * k
        for p in range(NB):
            wait_gather(g + p, p)
            start_scatter(g + p, p)
        for p in range(NB):
            wait_scatter(g + p, p)
            start_gather(g + NB + p, p)
        return carry

    lax.fori_loop(0, NCH // NB - 1, body, 0)

    g = NCH - NB
    for p in range(NB):
        wait_gather(g + p, p)
        start_scatter(g + p, p)
    for p in range(NB):
        wait_scatter(g + p, p)


def kernel(indices, tables):
    tbl = tables.reshape(F * V, D)
    tbl = jnp.concatenate([tbl, tbl, tbl, tbl], axis=0)
    flat_idx = indices + jnp.arange(F, dtype=jnp.int32)[None, :] * V
    rep = (jnp.arange(8, dtype=jnp.int32) % 4) * (F * V)
    perm_idx = (flat_idx.reshape(NBANDS, 8, F) + rep[None, :, None]
                ).transpose(0, 2, 1)
    idx3 = perm_idx.reshape(NW, NCH, CH)
    out = _sc_lookup(tbl, idx3)
    return out.transpose(0, 2, 1, 3).reshape(B, F * D)


# speedup vs baseline: 1.3192x; 1.3192x over previous
import functools

import jax
import jax.numpy as jnp
from jax import lax
from jax.experimental import pallas as pl
from jax.experimental.pallas import tpu as pltpu
from jax.experimental.pallas import tpu_sc as plsc

B, F, V, D = 16384, 26, 8, 128
NC, NS = 2, 16
NW = NC * NS
ROWS = B * F
RPW = ROWS // NW
CH = 128
NCH = RPW // CH
NB = 4
NBANDS = B // 8
BPB = F * 8
LANES = 16
NSL = RPW // LANES
SPB = BPB // LANES


@functools.partial(
    pl.kernel,
    out_type=jax.ShapeDtypeStruct((NBANDS, F, 8, D), jnp.float32),
    mesh=plsc.VectorSubcoreMesh(core_axis_name="c", subcore_axis_name="s"),
    compiler_params=pltpu.CompilerParams(needs_layout_passes=False),
    scratch_types=(
        [pltpu.VMEM((RPW,), jnp.int32),
         pltpu.VMEM((NCH, CH), jnp.int32)]
        + [pltpu.VMEM((CH, D), jnp.float32) for _ in range(NB)]
        + [pltpu.VMEM_SHARED((F * V, D), jnp.float32)]
        + [pltpu.SemaphoreType.DMA for _ in range(2 * NB)]
    ),
)
def _sc_lookup(tbl_hbm, idx_hbm, out_4d, raw_v, idx_v, *rest):
    out_hbm = out_4d.reshape(ROWS, D)
    bufs = rest[:NB]
    tbl_sh = rest[NB]
    gsem = rest[NB + 1:2 * NB + 1]
    ssem = rest[2 * NB + 1:]

    wid = lax.axis_index("s") * NC + lax.axis_index("c")

    @pl.when(lax.axis_index("s") == 0)
    def _stage_table():
        pltpu.sync_copy(tbl_hbm, tbl_sh)

    pltpu.sync_copy(idx_hbm.at[wid], raw_v)

    def permute(j, carry):
        band = j // SPB
        q = lax.iota(jnp.int32, LANES) + (j % SPB) * LANES
        f = q >> 3
        i = q & 7
        src = band * BPB + i * F + f
        vals = plsc.load_gather(raw_v, [src])
        idx_v[j // 8, pl.ds((j % 8) * LANES, LANES)] = vals + (f << 3)
        return carry

    lax.fori_loop(0, NSL, permute, 0)
    plsc.subcore_barrier()

    base = wid * RPW

    def start_gather(g, p):
        pltpu.async_copy(tbl_sh.at[idx_v.at[g]], bufs[p], gsem[p])

    def wait_gather(g, p):
        pltpu.make_async_copy(tbl_sh.at[idx_v.at[g]], bufs[p], gsem[p]).wait()

    def start_scatter(g, p):
        pltpu.async_copy(bufs[p], out_hbm.at[pl.ds(base + g * CH, CH)], ssem[p])

    def wait_scatter(g, p):
        pltpu.make_async_copy(bufs[p], out_hbm.at[pl.ds(base + g * CH, CH)],
                              ssem[p]).wait()

    for p in range(NB):
        start_gather(p, p)

    def body(k, carry):
        g = NB * k
        for p in range(NB):
            wait_gather(g + p, p)
            start_scatter(g + p, p)
        for p in range(NB):
            wait_scatter(g + p, p)
            start_gather(g + NB + p, p)
        return carry

    lax.fori_loop(0, NCH // NB - 1, body, 0)

    g = NCH - NB
    for p in range(NB):
        wait_gather(g + p, p)
        start_scatter(g + p, p)
    for p in range(NB):
        wait_scatter(g + p, p)


def kernel(indices, tables):
    tbl = tables.reshape(F * V, D)
    idx2 = indices.reshape(NW, RPW)
    out = _sc_lookup(tbl, idx2)
    return out.transpose(0, 2, 1, 3).reshape(B, F * D)


# speedup vs baseline: 1.3578x; 1.0293x over previous
import functools

import jax
import jax.numpy as jnp
from jax import lax
from jax.experimental import pallas as pl
from jax.experimental.pallas import tpu as pltpu
from jax.experimental.pallas import tpu_sc as plsc

B, F, V, D = 16384, 26, 8, 128
NC, NS = 2, 16
NW = NC * NS
ROWS = B * F
RPW = ROWS // NW
CH = 128
NCH = RPW // CH
NB = 4
NBANDS = B // 8
BPB = F * 8
LANES = 16
NSL = RPW // LANES
SPB = BPB // LANES


@functools.partial(
    pl.kernel,
    out_type=jax.ShapeDtypeStruct((NBANDS, F, 8, D), jnp.float32),
    mesh=plsc.VectorSubcoreMesh(core_axis_name="c", subcore_axis_name="s"),
    compiler_params=pltpu.CompilerParams(needs_layout_passes=False),
    scratch_types=(
        [pltpu.VMEM((RPW,), jnp.int32),
         pltpu.VMEM((NCH, CH), jnp.int32)]
        + [pltpu.VMEM((CH, D), jnp.float32) for _ in range(NB)]
        + [pltpu.VMEM_SHARED((F * V, D), jnp.float32)]
        + [pltpu.SemaphoreType.DMA for _ in range(2 * NB)]
    ),
)
def _sc_lookup(tbl_hbm, idx_hbm, out_4d, raw_v, idx_v, *rest):
    out_hbm = out_4d.reshape(ROWS, D)
    bufs = rest[:NB]
    tbl_sh = rest[NB]
    gsem = rest[NB + 1:2 * NB + 1]
    ssem = rest[2 * NB + 1:]

    wid = lax.axis_index("s") * NC + lax.axis_index("c")

    @pl.when(lax.axis_index("s") == 0)
    def _stage_table():
        pltpu.sync_copy(tbl_hbm, tbl_sh)

    pltpu.sync_copy(idx_hbm.at[wid], raw_v)

    def permute(j, carry):
        band = j // SPB
        q = lax.iota(jnp.int32, LANES) + (j % SPB) * LANES
        f = q >> 3
        i = q & 7
        src = band * BPB + i * F + f
        vals = plsc.load_gather(raw_v, [src])
        idx_v[j // 8, pl.ds((j % 8) * LANES, LANES)] = vals + (f << 3)
        return carry

    lax.fori_loop(0, 8 * NB, permute, 0)
    plsc.subcore_barrier()

    base = wid * RPW

    def start_gather(g, p):
        pltpu.async_copy(tbl_sh.at[idx_v.at[g]], bufs[p], gsem[p])

    def wait_gather(g, p):
        pltpu.make_async_copy(tbl_sh.at[idx_v.at[g]], bufs[p], gsem[p]).wait()

    def start_scatter(g, p):
        pltpu.async_copy(bufs[p], out_hbm.at[pl.ds(base + g * CH, CH)], ssem[p])

    def wait_scatter(g, p):
        pltpu.make_async_copy(bufs[p], out_hbm.at[pl.ds(base + g * CH, CH)],
                              ssem[p]).wait()

    for p in range(NB):
        start_gather(p, p)

    def body(k, carry):
        g = NB * k
        lax.fori_loop(8 * (g + NB), 8 * (g + 2 * NB), permute, 0)
        for p in range(NB):
            wait_gather(g + p, p)
            start_scatter(g + p, p)
        for p in range(NB):
            wait_scatter(g + p, p)
            start_gather(g + NB + p, p)
        return carry

    lax.fori_loop(0, NCH // NB - 1, body, 0)

    g = NCH - NB
    for p in range(NB):
        wait_gather(g + p, p)
        start_scatter(g + p, p)
    for p in range(NB):
        wait_scatter(g + p, p)


def kernel(indices, tables):
    tbl = tables.reshape(F * V, D)
    idx2 = indices.reshape(NW, RPW)
    out = _sc_lookup(tbl, idx2)
    return out.transpose(0, 2, 1, 3).reshape(B, F * D)


# speedup vs baseline: 1.3629x; 1.0037x over previous
import functools

import jax
import jax.numpy as jnp
from jax import lax
from jax.experimental import pallas as pl
from jax.experimental.pallas import tpu as pltpu
from jax.experimental.pallas import tpu_sc as plsc

B, F, V, D = 16384, 26, 8, 128
NC, NS = 2, 16
NW = NC * NS
ROWS = B * F
RPW = ROWS // NW
CH = 128
NCH = RPW // CH
NB = 4
NBANDS = B // 8
BPB = F * 8
LANES = 16
NSL = RPW // LANES
SPB = BPB // LANES


@functools.partial(
    pl.kernel,
    out_type=jax.ShapeDtypeStruct((NBANDS, F, 8, D), jnp.float32),
    mesh=plsc.VectorSubcoreMesh(core_axis_name="c", subcore_axis_name="s"),
    compiler_params=pltpu.CompilerParams(needs_layout_passes=False),
    scratch_types=(
        [pltpu.VMEM((RPW,), jnp.int32),
         pltpu.VMEM((NCH, CH), jnp.int32)]
        + [pltpu.VMEM((NB * CH, D), jnp.float32)]
        + [pltpu.VMEM_SHARED((F * V, D), jnp.float32)]
        + [pltpu.SemaphoreType.DMA for _ in range(NB + 2)]
    ),
)
def _sc_lookup(tbl_hbm, idx_hbm, out_4d, raw_v, idx_v, *rest):
    out_hbm = out_4d.reshape(ROWS, D)
    ring = rest[0]
    tbl_sh = rest[1]
    gsem = rest[2:2 + NB]
    ssem = rest[2 + NB:]

    wid = lax.axis_index("s") * NC + lax.axis_index("c")

    @pl.when(lax.axis_index("s") == 0)
    def _stage_table():
        pltpu.sync_copy(tbl_hbm, tbl_sh)

    pltpu.sync_copy(idx_hbm.at[wid], raw_v)

    def permute(j, carry):
        band = j // SPB
        q = lax.iota(jnp.int32, LANES) + (j % SPB) * LANES
        f = q >> 3
        i = q & 7
        src = band * BPB + i * F + f
        vals = plsc.load_gather(raw_v, [src])
        idx_v[j // 8, pl.ds((j % 8) * LANES, LANES)] = vals + (f << 3)
        return carry

    lax.fori_loop(0, 8 * NB, permute, 0)
    plsc.subcore_barrier()

    base = wid * RPW

    def slot(p):
        return ring.at[pl.ds(p * CH, CH)]

    def pair(h):
        return ring.at[pl.ds(h * 2 * CH, 2 * CH)]

    def start_gather(g, p):
        pltpu.async_copy(tbl_sh.at[idx_v.at[g]], slot(p), gsem[p])

    def wait_gather(g, p):
        pltpu.make_async_copy(tbl_sh.at[idx_v.at[g]], slot(p), gsem[p]).wait()

    def start_scatter2(g, h):
        pltpu.async_copy(pair(h), out_hbm.at[pl.ds(base + g * CH, 2 * CH)],
                         ssem[h])

    def wait_scatter2(g, h):
        pltpu.make_async_copy(pair(h), out_hbm.at[pl.ds(base + g * CH, 2 * CH)],
                              ssem[h]).wait()

    for p in range(NB):
        start_gather(p, p)

    def body(k, carry):
        g = NB * k
        lax.fori_loop(8 * (g + NB), 8 * (g + 2 * NB), permute, 0)
        for h in range(2):
            wait_gather(g + 2 * h, 2 * h)
            wait_gather(g + 2 * h + 1, 2 * h + 1)
            start_scatter2(g + 2 * h, h)
        for h in range(2):
            wait_scatter2(g + 2 * h, h)
            start_gather(g + NB + 2 * h, 2 * h)
            start_gather(g + NB + 2 * h + 1, 2 * h + 1)
        return carry

    lax.fori_loop(0, NCH // NB - 1, body, 0)

    g = NCH - NB
    for h in range(2):
        wait_gather(g + 2 * h, 2 * h)
        wait_gather(g + 2 * h + 1, 2 * h + 1)
        start_scatter2(g + 2 * h, h)
    for h in range(2):
        wait_scatter2(g + 2 * h, h)


def kernel(indices, tables):
    tbl = tables.reshape(F * V, D)
    idx2 = indices.reshape(NW, RPW)
    out = _sc_lookup(tbl, idx2)
    return out.transpose(0, 2, 1, 3).reshape(B, F * D)
